# Initial kernel scaffold; baseline (speedup 1.0000x reference)
import jax
import jax.numpy as jnp
from jax.experimental import pallas as pl
from jax.experimental.pallas import tpu as pltpu
from jax.experimental.pallas import tpu_sc as plsc

DIM = 64
WINDOW = 512


def kernel(head, tail, table):
    B, L = head.shape
    n = B * L
    head_i = head.reshape(1, n).astype(jnp.int32)
    tail_i = tail.reshape(1, n).astype(jnp.int32)

    mesh = plsc.VectorSubcoreMesh(core_axis_name="c", subcore_axis_name="s")
    out_sds = jax.ShapeDtypeStruct((n, DIM), table.dtype)

    @pl.kernel(out_type=(out_sds, out_sds), mesh=mesh)
    def gather_kernel(table_hbm, hi_hbm, ti_hbm, ho_hbm, to_hbm):
        def body(hi_vmem, ti_vmem, ho_vmem, to_vmem):
            pltpu.sync_copy(table_hbm.at[hi_vmem.at[0]], ho_vmem)
            pltpu.sync_copy(table_hbm.at[ti_vmem.at[0]], to_vmem)

        pltpu.emit_pipeline(
            body,
            grid=(n // WINDOW,),
            in_specs=[
                pl.BlockSpec((1, WINDOW), lambda i: (0, i)),
                pl.BlockSpec((1, WINDOW), lambda i: (0, i)),
            ],
            out_specs=[
                pl.BlockSpec((WINDOW, DIM), lambda i: (i, 0)),
                pl.BlockSpec((WINDOW, DIM), lambda i: (i, 0)),
            ],
            core_axis_name=("c", "s"),
            dimension_semantics=(pltpu.PARALLEL,),
        )(hi_hbm, ti_hbm, ho_hbm, to_hbm)

    ho, to = gather_kernel(table, head_i, tail_i)
    return ho.reshape(B, L, DIM), to.reshape(B, L, DIM)


# speedup vs baseline: 1.0362x; 1.0362x over previous
import jax
import jax.numpy as jnp
from jax.experimental import pallas as pl
from jax.experimental.pallas import tpu as pltpu
from jax.experimental.pallas import tpu_sc as plsc

DIM = 64
WINDOW = 256


def kernel(head, tail, table):
    B, L = head.shape
    n = B * L
    head_i = head.reshape(1, n).astype(jnp.int32)
    tail_i = tail.reshape(1, n).astype(jnp.int32)

    mesh = plsc.VectorSubcoreMesh(core_axis_name="c", subcore_axis_name="s")
    out_sds = jax.ShapeDtypeStruct((n, DIM), table.dtype)

    @pl.kernel(
        out_type=(out_sds, out_sds),
        mesh=mesh,
        compiler_params=pltpu.CompilerParams(use_tc_tiling_on_sc=False),
    )
    def gather_kernel(table_hbm, hi_hbm, ti_hbm, ho_hbm, to_hbm):
        def body(hi_vmem, ti_vmem, ho_vmem, to_vmem):
            pltpu.sync_copy(table_hbm.at[hi_vmem.at[0]], ho_vmem)
            pltpu.sync_copy(table_hbm.at[ti_vmem.at[0]], to_vmem)

        pltpu.emit_pipeline(
            body,
            grid=(n // WINDOW,),
            in_specs=[
                pl.BlockSpec((1, WINDOW), lambda i: (0, i)),
                pl.BlockSpec((1, WINDOW), lambda i: (0, i)),
            ],
            out_specs=[
                pl.BlockSpec((WINDOW, DIM), lambda i: (i, 0)),
                pl.BlockSpec((WINDOW, DIM), lambda i: (i, 0)),
            ],
            core_axis_name=("c", "s"),
            dimension_semantics=(pltpu.PARALLEL,),
        )(hi_hbm, ti_hbm, ho_hbm, to_hbm)

    ho, to = gather_kernel(table, head_i, tail_i)
    return ho.reshape(B, L, DIM), to.reshape(B, L, DIM)


# speedup vs baseline: 1.1311x; 1.0916x over previous
import jax
from jax import lax
import jax.numpy as jnp
from jax.experimental import pallas as pl
from jax.experimental.pallas import tpu as pltpu
from jax.experimental.pallas import tpu_sc as plsc

DIM = 64
PAD = 128
NS = 16
CB = 8


def kernel(head, tail, table):
    B, L = head.shape
    n = B * L
    rw = B // NS
    rwl = rw * L
    cbl = CB * L
    C = rw // CB
    head_i = head.reshape(n).astype(jnp.int32)
    tail_i = tail.reshape(n).astype(jnp.int32)
    tab128 = jnp.pad(table, ((0, 0), (0, PAD - DIM)))

    mesh = plsc.VectorSubcoreMesh(core_axis_name="c", subcore_axis_name="s")
    out_sds = jax.ShapeDtypeStruct((B, L, DIM), table.dtype)

    @pl.kernel(
        out_type=(out_sds, out_sds),
        mesh=mesh,
        scratch_types=[
            pltpu.VMEM((rwl,), jnp.int32),
            pltpu.VMEM((cbl, PAD), jnp.float32),
            pltpu.VMEM((cbl, PAD), jnp.float32),
            pltpu.VMEM((CB, L, DIM), jnp.float32),
            pltpu.VMEM((CB, L, DIM), jnp.float32),
            pltpu.SemaphoreType.DMA,
            pltpu.SemaphoreType.DMA,
            pltpu.SemaphoreType.DMA,
            pltpu.SemaphoreType.DMA,
        ],
    )
    def gather_kernel(tab_hbm, hi_hbm, ti_hbm, ho_hbm, to_hbm,
                      idx_v, g0, g1, o0, o1, gs0, gs1, ws0, ws1):
        wid = lax.axis_index("c") * NS + lax.axis_index("s")

        def gather(c, gbuf, gsem):
            return pltpu.async_copy(
                tab_hbm.at[idx_v.at[pl.ds(c * cbl, cbl)]], gbuf, gsem)

        def compact(gbuf, obuf):
            @pl.loop(0, CB)
            def _(b):
                @pl.loop(0, L)
                def _(j):
                    r = b * L + j
                    for k in range(DIM // 16):
                        obuf[b, j, pl.ds(k * 16, 16)] = (
                            gbuf[r, pl.ds(k * 16, 16)])

        def run(idx_hbm, out_hbm, w):
            ibase = w * rwl
            obase = w * rw
            pltpu.sync_copy(idx_hbm.at[pl.ds(ibase, rwl)], idx_v)
            gather(0, g0, gs0)
            gather(1, g1, gs1)

            def stage(c, gbuf, gsem, obuf, wsem):
                pltpu.make_async_copy(
                    tab_hbm.at[idx_v.at[pl.ds(c * cbl, cbl)]], gbuf,
                    gsem).wait()

                @pl.when(c >= 2)
                def _():
                    pltpu.make_async_copy(
                        obuf, out_hbm.at[pl.ds(obase + (c - 2) * CB, CB)],
                        wsem).wait()

                compact(gbuf, obuf)
                pltpu.async_copy(
                    obuf, out_hbm.at[pl.ds(obase + c * CB, CB)], wsem)

                @pl.when(c + 2 < C)
                def _():
                    gather(c + 2, gbuf, gsem)

            @pl.loop(0, C, step=2)
            def _(c):
                stage(c, g0, gs0, o0, ws0)
                stage(c + 1, g1, gs1, o1, ws1)

            pltpu.make_async_copy(
                o0, out_hbm.at[pl.ds(obase + (C - 2) * CB, CB)], ws0).wait()
            pltpu.make_async_copy(
                o1, out_hbm.at[pl.ds(obase + (C - 1) * CB, CB)], ws1).wait()

        @pl.when(wid < NS)
        def _():
            run(hi_hbm, ho_hbm, wid)

        @pl.when(wid >= NS)
        def _():
            run(ti_hbm, to_hbm, wid - NS)

    ho, to = gather_kernel(tab128, head_i, tail_i)
    return ho, to


# speedup vs baseline: 1.1800x; 1.0432x over previous
import jax
from jax import lax
import jax.numpy as jnp
from jax.experimental import pallas as pl
from jax.experimental.pallas import tpu as pltpu
from jax.experimental.pallas import tpu_sc as plsc

DIM = 64
PAD = 128
NS = 16
CB = 8
NB = 2048


def _transpose_pad(table):
    V = table.shape[0]
    tab_t = jnp.transpose(table)

    def body(t_ref, o_ref):
        x = t_ref[...]
        xt = jnp.transpose(x)
        o_ref[...] = jnp.concatenate([xt, jnp.zeros_like(xt)], axis=1)

    return pl.pallas_call(
        body,
        grid=(pl.cdiv(V, NB),),
        in_specs=[pl.BlockSpec((DIM, NB), lambda i: (0, i))],
        out_specs=pl.BlockSpec((NB, PAD), lambda i: (i, 0)),
        out_shape=jax.ShapeDtypeStruct((V, PAD), jnp.float32),
    )(tab_t)


def kernel(head, tail, table):
    B, L = head.shape
    n = B * L
    rw = B // NS
    rwl = rw * L
    cbl = CB * L
    C = rw // CB
    head_i = head.reshape(n).astype(jnp.int32)
    tail_i = tail.reshape(n).astype(jnp.int32)
    tab128 = _transpose_pad(table)

    mesh = plsc.VectorSubcoreMesh(core_axis_name="c", subcore_axis_name="s")
    out_sds = jax.ShapeDtypeStruct((B, L, DIM), table.dtype)

    @pl.kernel(
        out_type=(out_sds, out_sds),
        mesh=mesh,
        scratch_types=[
            pltpu.VMEM((rwl,), jnp.int32),
            pltpu.VMEM((cbl, PAD), jnp.float32),
            pltpu.VMEM((cbl, PAD), jnp.float32),
            pltpu.VMEM((CB, L, DIM), jnp.float32),
            pltpu.VMEM((CB, L, DIM), jnp.float32),
            pltpu.SemaphoreType.DMA,
            pltpu.SemaphoreType.DMA,
            pltpu.SemaphoreType.DMA,
            pltpu.SemaphoreType.DMA,
        ],
    )
    def gather_kernel(tab_hbm, hi_hbm, ti_hbm, ho_hbm, to_hbm,
                      idx_v, g0, g1, o0, o1, gs0, gs1, ws0, ws1):
        wid = lax.axis_index("c") * NS + lax.axis_index("s")

        def gather(c, gbuf, gsem):
            return pltpu.async_copy(
                tab_hbm.at[idx_v.at[pl.ds(c * cbl, cbl)]], gbuf, gsem)

        def compact(gbuf, obuf):
            @pl.loop(0, CB)
            def _(b):
                @pl.loop(0, L)
                def _(j):
                    r = b * L + j
                    for k in range(DIM // 16):
                        obuf[b, j, pl.ds(k * 16, 16)] = (
                            gbuf[r, pl.ds(k * 16, 16)])

        def run(idx_hbm, out_hbm, w):
            ibase = w * rwl
            obase = w * rw
            pltpu.sync_copy(idx_hbm.at[pl.ds(ibase, rwl)], idx_v)
            gather(0, g0, gs0)
            gather(1, g1, gs1)

            def stage(c, gbuf, gsem, obuf, wsem):
                pltpu.make_async_copy(
                    tab_hbm.at[idx_v.at[pl.ds(c * cbl, cbl)]], gbuf,
                    gsem).wait()

                @pl.when(c >= 2)
                def _():
                    pltpu.make_async_copy(
                        obuf, out_hbm.at[pl.ds(obase + (c - 2) * CB, CB)],
                        wsem).wait()

                compact(gbuf, obuf)
                pltpu.async_copy(
                    obuf, out_hbm.at[pl.ds(obase + c * CB, CB)], wsem)

                @pl.when(c + 2 < C)
                def _():
                    gather(c + 2, gbuf, gsem)

            @pl.loop(0, C, step=2)
            def _(c):
                stage(c, g0, gs0, o0, ws0)
                stage(c + 1, g1, gs1, o1, ws1)

            pltpu.make_async_copy(
                o0, out_hbm.at[pl.ds(obase + (C - 2) * CB, CB)], ws0).wait()
            pltpu.make_async_copy(
                o1, out_hbm.at[pl.ds(obase + (C - 1) * CB, CB)], ws1).wait()

        @pl.when(wid < NS)
        def _():
            run(hi_hbm, ho_hbm, wid)

        @pl.when(wid >= NS)
        def _():
            run(ti_hbm, to_hbm, wid - NS)

    ho, to = gather_kernel(tab128, head_i, tail_i)
    return ho, to
